# Initial kernel scaffold; baseline (speedup 1.0000x reference)
import jax
import jax.numpy as jnp
from jax.experimental import pallas as pl


def kernel(x, W, b):
    raise NotImplementedError("write your pallas kernel here")


# speedup vs baseline: 11.6823x; 11.6823x over previous
import functools

import jax
import jax.numpy as jnp
from jax import lax
from jax.experimental import pallas as pl
from jax.experimental.pallas import tpu as pltpu
from jax.experimental.pallas import tpu_sc as plsc

NBINS = 256
NOUT = 128
NROWS = 16
ROWLEN = 1_000_000
NCORES = 2
NSUB = 16
NWORKERS = NCORES * NSUB
PER_W = (NROWS * ROWLEN) // NWORKERS
CHUNK = 50_000
NCH = PER_W // CHUNK
VPB = CHUNK // 16
UNROLL = 25


def _sc_body(x_hbm, hist_hbm, mm_hbm, buf_a, buf_b, hist_v, tmp_a, tmp_b,
             sh_mm, sem_a, sem_b):
    c = lax.axis_index("c")
    s = lax.axis_index("s")
    wid = c * NSUB + s
    row = wid // 2
    half = s & 1
    base = wid * PER_W

    def start(chunk, buf, sem):
        pltpu.make_async_copy(
            x_hbm.at[pl.ds(base + chunk * CHUNK, CHUNK)], buf, sem).start()

    def wait(chunk, buf, sem):
        pltpu.make_async_copy(
            x_hbm.at[pl.ds(base + chunk * CHUNK, CHUNK)], buf, sem).wait()

    def run_pass(process_chunk, carry_init):
        start(0, buf_a, sem_a)
        start(1, buf_b, sem_b)

        def outer(i, carry):
            for b, (buf, sem) in enumerate(((buf_a, sem_a), (buf_b, sem_b))):
                chunk = 2 * i + b
                wait(chunk, buf, sem)
                carry = process_chunk(buf, carry)

                @pl.when(chunk + 2 < NCH)
                def _():
                    start(chunk + 2, buf, sem)
            return carry

        return lax.fori_loop(0, NCH // 2, outer, carry_init)

    def p1_chunk(buf, carry):
        def inner(j, cr):
            mn_v, mx_v = cr
            off = j * (16 * UNROLL)
            for u in range(UNROLL):
                v = buf[pl.ds(off + u * 16, 16)]
                mn_v = jnp.minimum(mn_v, v)
                mx_v = jnp.maximum(mx_v, v)
            return (mn_v, mx_v)

        return lax.fori_loop(0, VPB // UNROLL, inner, carry)

    big = jnp.full((16,), jnp.inf, jnp.float32)
    mn_v, mx_v = run_pass(p1_chunk, (big, -big))

    tmp_a[...] = mn_v
    tmp_b[...] = mx_v
    pltpu.sync_copy(tmp_a, sh_mm.at[s, 0])
    pltpu.sync_copy(tmp_b, sh_mm.at[s, 1])
    plsc.subcore_barrier()
    partner = jnp.bitwise_xor(s, 1)
    pltpu.sync_copy(sh_mm.at[partner, 0], tmp_a)
    pltpu.sync_copy(sh_mm.at[partner, 1], tmp_b)
    mn_v = jnp.minimum(mn_v, tmp_a[...])
    mx_v = jnp.maximum(mx_v, tmp_b[...])

    lane = lax.iota(jnp.int32, 16)
    for sh in (8, 4, 2, 1):
        perm = jnp.bitwise_xor(lane, sh)
        tmp_a[...] = mn_v
        tmp_b[...] = mx_v
        mn_v = jnp.minimum(mn_v, plsc.load_gather(tmp_a, [perm]))
        mx_v = jnp.maximum(mx_v, plsc.load_gather(tmp_b, [perm]))
    mn_b = mn_v
    mx_b = mx_v
    scale_b = float(NBINS) / (mx_b - mn_b)

    @pl.when(half == 0)
    def _():
        mmv = jnp.where(lane == 0, mn_b, jnp.where(lane == 1, mx_b, 0.0))
        tmp_a[...] = mmv
        pltpu.sync_copy(tmp_a, mm_hbm.at[row])

    zv = jnp.zeros((16,), jnp.float32)

    def zero(i, cr):
        hist_v[i, :] = zv
        return cr

    lax.fori_loop(0, NBINS, zero, 0)

    one_v = jnp.full((16,), 1.0, jnp.float32)
    lim_v = jnp.full((16,), NBINS - 1, jnp.int32)
    zero_i = jnp.zeros((16,), jnp.int32)

    def p2_chunk(buf, carry):
        def inner(j, cr):
            off = j * (16 * UNROLL)
            for u in range(UNROLL):
                v = buf[pl.ds(off + u * 16, 16)]
                t = (v - mn_b) * scale_b
                idx = t.astype(jnp.int32)
                idx = jnp.maximum(jnp.minimum(idx, lim_v), zero_i)
                plsc.addupdate_scatter(hist_v, [idx, lane], one_v)
            return cr

        return lax.fori_loop(0, VPB // UNROLL, inner, carry)

    run_pass(p2_chunk, 0)

    pltpu.sync_copy(hist_v, hist_hbm.at[wid])


_sc_hist = functools.partial(
    pl.kernel,
    out_type=[
        jax.ShapeDtypeStruct((NWORKERS, NBINS, 16), jnp.float32),
        jax.ShapeDtypeStruct((NROWS, 16), jnp.float32),
    ],
    mesh=plsc.VectorSubcoreMesh(
        core_axis_name="c", subcore_axis_name="s",
        num_cores=NCORES, num_subcores=NSUB),
    compiler_params=pltpu.CompilerParams(
        needs_layout_passes=False, use_tc_tiling_on_sc=False),
    scratch_types=[
        pltpu.VMEM((CHUNK,), jnp.float32),
        pltpu.VMEM((CHUNK,), jnp.float32),
        pltpu.VMEM((NBINS, 16), jnp.float32),
        pltpu.VMEM((16,), jnp.float32),
        pltpu.VMEM((16,), jnp.float32),
        pltpu.VMEM_SHARED((NSUB, 2, 16), jnp.float32),
        pltpu.SemaphoreType.DMA,
        pltpu.SemaphoreType.DMA,
    ],
)(_sc_body)


def _tc_body(hist_ref, mm_ref, wc_ref, wb_ref, b_ref, out_ref):
    h = hist_ref[...].reshape(NROWS, 2, NBINS, 16)
    counts = jnp.sum(h, axis=(1, 3))
    total = jnp.sum(counts, axis=1, keepdims=True)
    counts = counts / total
    mn = mm_ref[:, 0:1]
    mx = mm_ref[:, 1:2]
    k = lax.broadcasted_iota(jnp.int32, (NROWS, NBINS + 1), 1).astype(jnp.float32)
    bounds = mn + (mx - mn) * (k * (1.0 / NBINS))
    acc = lax.dot_general(counts, wc_ref[...], (((1,), (0,)), ((), ())),
                          preferred_element_type=jnp.float32)
    acc += lax.dot_general(bounds, wb_ref[...], (((1,), (0,)), ((), ())),
                           preferred_element_type=jnp.float32)
    out_ref[...] = acc + b_ref[...][None, :]


def _tc_linear(hist, mm, wc, wb, b):
    return pl.pallas_call(
        _tc_body,
        out_shape=jax.ShapeDtypeStruct((NROWS, NOUT), jnp.float32),
    )(hist, mm, wc, wb, b)


def kernel(x, W, b):
    xf = x.reshape(-1)
    hist, mm = _sc_hist(xf)
    wc = W[:, :NBINS].T
    wb = W[:, NBINS:].T
    return _tc_linear(hist, mm, wc, wb, b)


# speedup vs baseline: 15.3949x; 1.3178x over previous
import functools

import jax
import jax.numpy as jnp
from jax import lax
from jax.experimental import pallas as pl
from jax.experimental.pallas import tpu as pltpu
from jax.experimental.pallas import tpu_sc as plsc

NBINS = 256
NOUT = 128
NROWS = 16
ROWLEN = 1_000_000
NCORES = 2
NSUB = 16
NWORKERS = NCORES * NSUB
PER_W = (NROWS * ROWLEN) // NWORKERS
CHUNK = 50_000
NCH = PER_W // CHUNK
VPB = CHUNK // 16
UNROLL = 25


def _sc_body(x_hbm, hist_hbm, mm_hbm, buf_a, buf_b, hist_v, tmp_a, tmp_b,
             sh_mm, sem_a, sem_b):
    c = lax.axis_index("c")
    s = lax.axis_index("s")
    wid = c * NSUB + s
    row = wid // 2
    base = wid * PER_W

    def start(chunk, buf, sem):
        pltpu.make_async_copy(
            x_hbm.at[pl.ds(base + chunk * CHUNK, CHUNK)], buf, sem).start()

    def wait(chunk, buf, sem):
        pltpu.make_async_copy(
            x_hbm.at[pl.ds(base + chunk * CHUNK, CHUNK)], buf, sem).wait()

    def run_pass(process_chunk, carry_init):
        start(0, buf_a, sem_a)
        start(1, buf_b, sem_b)

        def outer(i, carry):
            for b, (buf, sem) in enumerate(((buf_a, sem_a), (buf_b, sem_b))):
                chunk = 2 * i + b
                wait(chunk, buf, sem)
                carry = process_chunk(buf, carry)

                @pl.when(chunk + 2 < NCH)
                def _():
                    start(chunk + 2, buf, sem)
            return carry

        return lax.fori_loop(0, NCH // 2, outer, carry_init)

    def p1_chunk(buf, carry):
        def inner(j, cr):
            mn_v, mx_v = cr
            off = j * (16 * UNROLL)
            for u in range(UNROLL):
                v = buf[pl.ds(off + u * 16, 16)]
                mn_v = jnp.minimum(mn_v, v)
                mx_v = jnp.maximum(mx_v, v)
            return (mn_v, mx_v)

        return lax.fori_loop(0, VPB // UNROLL, inner, carry)

    big = jnp.full((16,), jnp.inf, jnp.float32)
    mn_v, mx_v = run_pass(p1_chunk, (big, -big))

    tmp_a[...] = mn_v
    tmp_b[...] = mx_v
    pltpu.sync_copy(tmp_a, sh_mm.at[s, 0])
    pltpu.sync_copy(tmp_b, sh_mm.at[s, 1])
    plsc.subcore_barrier()
    partner = jnp.bitwise_xor(s, 1)
    pltpu.sync_copy(sh_mm.at[partner, 0], tmp_a)
    pltpu.sync_copy(sh_mm.at[partner, 1], tmp_b)
    mn_v = jnp.minimum(mn_v, tmp_a[...])
    mx_v = jnp.maximum(mx_v, tmp_b[...])

    lane = lax.iota(jnp.int32, 16)
    for sh in (8, 4, 2, 1):
        perm = jnp.bitwise_xor(lane, sh)
        tmp_a[...] = mn_v
        tmp_b[...] = mx_v
        mn_v = jnp.minimum(mn_v, plsc.load_gather(tmp_a, [perm]))
        mx_v = jnp.maximum(mx_v, plsc.load_gather(tmp_b, [perm]))
    mn_b = mn_v
    mx_b = mx_v
    scale_b = float(NBINS) / (mx_b - mn_b)

    @pl.when((s & 1) == 0)
    def _():
        mmv = jnp.where(lane == 0, mn_b, jnp.where(lane == 1, mx_b, 0.0))
        tmp_a[...] = mmv
        pltpu.sync_copy(tmp_a, mm_hbm.at[row])

    zv = jnp.zeros((16,), jnp.float32)

    def zero(i, cr):
        hist_v[i, :] = zv
        return cr

    lax.fori_loop(0, NBINS, zero, 0)

    one_v = jnp.full((16,), 1.0, jnp.float32)
    lim_v = jnp.full((16,), NBINS - 1, jnp.int32)
    zero_i = jnp.zeros((16,), jnp.int32)

    def p2_chunk(buf, carry):
        def inner(j, cr):
            off = j * (16 * UNROLL)
            for u in range(UNROLL):
                v = buf[pl.ds(off + u * 16, 16)]
                t = (v - mn_b) * scale_b
                idx = t.astype(jnp.int32)
                idx = jnp.maximum(jnp.minimum(idx, lim_v), zero_i)
                plsc.addupdate_scatter(hist_v, [idx, lane], one_v)
            return cr

        return lax.fori_loop(0, VPB // UNROLL, inner, carry)

    run_pass(p2_chunk, 0)

    pltpu.sync_copy(hist_v, hist_hbm.at[wid])


_sc_hist = functools.partial(
    pl.kernel,
    out_type=[
        jax.ShapeDtypeStruct((NWORKERS, NBINS, 16), jnp.float32),
        jax.ShapeDtypeStruct((NROWS, 16), jnp.float32),
    ],
    mesh=plsc.VectorSubcoreMesh(
        core_axis_name="c", subcore_axis_name="s",
        num_cores=NCORES, num_subcores=NSUB),
    compiler_params=pltpu.CompilerParams(
        needs_layout_passes=False, use_tc_tiling_on_sc=False),
    scratch_types=[
        pltpu.VMEM((CHUNK,), jnp.float32),
        pltpu.VMEM((CHUNK,), jnp.float32),
        pltpu.VMEM((NBINS, 16), jnp.float32),
        pltpu.VMEM((16,), jnp.float32),
        pltpu.VMEM((16,), jnp.float32),
        pltpu.VMEM_SHARED((NSUB, 2, 16), jnp.float32),
        pltpu.SemaphoreType.DMA,
        pltpu.SemaphoreType.DMA,
    ],
)(_sc_body)


def _tc_body(hist_ref, mm_ref, wc_ref, wb_ref, b_ref, out_ref):
    h = hist_ref[...].reshape(NROWS, 2, NBINS, 16)
    counts = jnp.sum(h, axis=(1, 3))
    total = jnp.sum(counts, axis=1, keepdims=True)
    counts = counts / total
    mn = mm_ref[:, 0:1]
    mx = mm_ref[:, 1:2]
    k = lax.broadcasted_iota(jnp.int32, (NROWS, NBINS + 1), 1).astype(jnp.float32)
    bounds = mn + (mx - mn) * (k * (1.0 / NBINS))
    acc = lax.dot_general(counts, wc_ref[...], (((1,), (0,)), ((), ())),
                          preferred_element_type=jnp.float32)
    acc += lax.dot_general(bounds, wb_ref[...], (((1,), (0,)), ((), ())),
                           preferred_element_type=jnp.float32)
    out_ref[...] = acc + b_ref[...][None, :]


def _tc_linear(hist, mm, wc, wb, b):
    return pl.pallas_call(
        _tc_body,
        out_shape=jax.ShapeDtypeStruct((NROWS, NOUT), jnp.float32),
    )(hist, mm, wc, wb, b)


def kernel(x, W, b):
    xf = jnp.concatenate([x[i] for i in range(NROWS)])
    hist, mm = _sc_hist(xf)
    wc = W[:, :NBINS].T
    wb = W[:, NBINS:].T
    return _tc_linear(hist, mm, wc, wb, b)


# speedup vs baseline: 21.4228x; 1.3916x over previous
import functools

import jax
import jax.numpy as jnp
from jax import lax
from jax.experimental import pallas as pl
from jax.experimental.pallas import tpu as pltpu
from jax.experimental.pallas import tpu_sc as plsc

NBINS = 256
NOUT = 128
NROWS = 16
ROWLEN = 1_000_000
NCORES = 2
NSUB = 16
NWORKERS = NCORES * NSUB
PER_W = (NROWS * ROWLEN) // NWORKERS
CHUNK = 50_000
NCH = PER_W // CHUNK
VPB = CHUNK // 16
UNROLL = 25


def _sc_body(x_hbm, hist_hbm, mm_hbm, buf_a, buf_b, hist_v, tmp_a, tmp_b,
             sh_mm, sem_a, sem_b):
    c = lax.axis_index("c")
    s = lax.axis_index("s")
    wid = c * NSUB + s
    row = wid // 2
    base = wid * PER_W

    def start(chunk, buf, sem):
        pltpu.make_async_copy(
            x_hbm.at[pl.ds(base + chunk * CHUNK, CHUNK)], buf, sem).start()

    def wait(chunk, buf, sem):
        pltpu.make_async_copy(
            x_hbm.at[pl.ds(base + chunk * CHUNK, CHUNK)], buf, sem).wait()

    def run_pass(process_chunk, carry_init):
        start(0, buf_a, sem_a)
        start(1, buf_b, sem_b)

        def outer(i, carry):
            for b, (buf, sem) in enumerate(((buf_a, sem_a), (buf_b, sem_b))):
                chunk = 2 * i + b
                wait(chunk, buf, sem)
                carry = process_chunk(buf, carry)

                @pl.when(chunk + 2 < NCH)
                def _():
                    start(chunk + 2, buf, sem)
            return carry

        return lax.fori_loop(0, NCH // 2, outer, carry_init)

    def p1_chunk(buf, carry):
        def inner(j, cr):
            mn_v, mx_v = cr
            off = j * (16 * UNROLL)
            for u in range(UNROLL):
                v = buf[pl.ds(off + u * 16, 16)]
                mn_v = jnp.minimum(mn_v, v)
                mx_v = jnp.maximum(mx_v, v)
            return (mn_v, mx_v)

        return lax.fori_loop(0, VPB // UNROLL, inner, carry)

    big = jnp.full((16,), jnp.inf, jnp.float32)
    mn_v, mx_v = run_pass(p1_chunk, (big, -big))

    tmp_a[...] = mn_v
    tmp_b[...] = mx_v
    pltpu.sync_copy(tmp_a, sh_mm.at[s, 0])
    pltpu.sync_copy(tmp_b, sh_mm.at[s, 1])
    plsc.subcore_barrier()
    partner = jnp.bitwise_xor(s, 1)
    pltpu.sync_copy(sh_mm.at[partner, 0], tmp_a)
    pltpu.sync_copy(sh_mm.at[partner, 1], tmp_b)
    mn_v = jnp.minimum(mn_v, tmp_a[...])
    mx_v = jnp.maximum(mx_v, tmp_b[...])

    lane = lax.iota(jnp.int32, 16)
    for sh in (8, 4, 2, 1):
        perm = jnp.bitwise_xor(lane, sh)
        tmp_a[...] = mn_v
        tmp_b[...] = mx_v
        mn_v = jnp.minimum(mn_v, plsc.load_gather(tmp_a, [perm]))
        mx_v = jnp.maximum(mx_v, plsc.load_gather(tmp_b, [perm]))
    mn_b = mn_v
    mx_b = mx_v
    scale_b = float(NBINS) / (mx_b - mn_b)

    @pl.when((s & 1) == 0)
    def _():
        mmv = jnp.where(lane == 0, mn_b, jnp.where(lane == 1, mx_b, 0.0))
        tmp_a[...] = mmv
        pltpu.sync_copy(tmp_a, mm_hbm.at[row])

    zv = jnp.zeros((16,), jnp.float32)

    def zero(i, cr):
        hist_v[i, :] = zv
        return cr

    lax.fori_loop(0, NBINS, zero, 0)

    one_v = jnp.full((16,), 1.0, jnp.float32)
    lim_v = jnp.full((16,), NBINS - 1, jnp.int32)
    zero_i = jnp.zeros((16,), jnp.int32)

    def p2_chunk(buf, carry):
        @plsc.parallel_loop(0, VPB, unroll=5)
        def _(i):
            v = buf[pl.ds(i * 16, 16)]
            t = (v - mn_b) * scale_b
            idx = t.astype(jnp.int32)
            idx = jnp.maximum(jnp.minimum(idx, lim_v), zero_i)
            plsc.addupdate_scatter(hist_v, [idx, lane], one_v)

        return carry

    run_pass(p2_chunk, 0)

    pltpu.sync_copy(hist_v, hist_hbm.at[wid])


_sc_hist = functools.partial(
    pl.kernel,
    out_type=[
        jax.ShapeDtypeStruct((NWORKERS, NBINS, 16), jnp.float32),
        jax.ShapeDtypeStruct((NROWS, 16), jnp.float32),
    ],
    mesh=plsc.VectorSubcoreMesh(
        core_axis_name="c", subcore_axis_name="s",
        num_cores=NCORES, num_subcores=NSUB),
    compiler_params=pltpu.CompilerParams(
        needs_layout_passes=False, use_tc_tiling_on_sc=False),
    scratch_types=[
        pltpu.VMEM((CHUNK,), jnp.float32),
        pltpu.VMEM((CHUNK,), jnp.float32),
        pltpu.VMEM((NBINS, 16), jnp.float32),
        pltpu.VMEM((16,), jnp.float32),
        pltpu.VMEM((16,), jnp.float32),
        pltpu.VMEM_SHARED((NSUB, 2, 16), jnp.float32),
        pltpu.SemaphoreType.DMA,
        pltpu.SemaphoreType.DMA,
    ],
)(_sc_body)


def _tc_body(hist_ref, mm_ref, wc_ref, wb_ref, b_ref, out_ref):
    h = hist_ref[...].reshape(NROWS, 2, NBINS, 16)
    counts = jnp.sum(h, axis=(1, 3))
    total = jnp.sum(counts, axis=1, keepdims=True)
    counts = counts / total
    mn = mm_ref[:, 0:1]
    mx = mm_ref[:, 1:2]
    k = lax.broadcasted_iota(jnp.int32, (NROWS, NBINS + 1), 1).astype(jnp.float32)
    bounds = mn + (mx - mn) * (k * (1.0 / NBINS))
    acc = lax.dot_general(counts, wc_ref[...], (((1,), (0,)), ((), ())),
                          preferred_element_type=jnp.float32)
    acc += lax.dot_general(bounds, wb_ref[...], (((1,), (0,)), ((), ())),
                           preferred_element_type=jnp.float32)
    out_ref[...] = acc + b_ref[...][None, :]


def _tc_linear(hist, mm, wc, wb, b):
    return pl.pallas_call(
        _tc_body,
        out_shape=jax.ShapeDtypeStruct((NROWS, NOUT), jnp.float32),
    )(hist, mm, wc, wb, b)


def kernel(x, W, b):
    xf = jnp.concatenate([x[i] for i in range(NROWS)])
    hist, mm = _sc_hist(xf)
    wc = W[:, :NBINS].T
    wb = W[:, NBINS:].T
    return _tc_linear(hist, mm, wc, wb, b)


# speedup vs baseline: 123.4556x; 5.7628x over previous
import functools

import jax
import jax.numpy as jnp
from jax import lax
from jax.experimental import pallas as pl
from jax.experimental.pallas import tpu as pltpu
from jax.experimental.pallas import tpu_sc as plsc

NBINS = 256
NOUT = 128
NROWS = 16
ROWLEN = 1_000_000
NCORES = 2
NSUB = 16
NWORKERS = NCORES * NSUB
CHUNK = 32_768
NCB = (ROWLEN + CHUNK - 1) // CHUNK
K0 = 15
TAIL = ROWLEN - 30 * CHUNK
VPB = CHUNK // 16
VPT = TAIL // 16
XFLEN = NCB * NROWS * CHUNK


def _linz_body(x_ref, o_ref):
    for r in range(NROWS):
        o_ref[pl.ds(r * CHUNK, CHUNK)] = x_ref[r, :]


_linz = pl.pallas_call(
    _linz_body,
    grid=(NCB,),
    in_specs=[pl.BlockSpec((NROWS, CHUNK), lambda j: (0, j))],
    out_specs=pl.BlockSpec((NROWS * CHUNK,), lambda j: (j,)),
    out_shape=jax.ShapeDtypeStruct((XFLEN,), jnp.float32),
)


def _sc_body(x_hbm, hist_hbm, mm_hbm, buf_a, buf_b, hist_v, tmp_a, tmp_b,
             sh_mm, sem_a, sem_b):
    c = lax.axis_index("c")
    s = lax.axis_index("s")
    wid = c * NSUB + s
    row = wid // 2
    half = s & 1
    jbase = half * K0

    def chunk_slice(k):
        return x_hbm.at[pl.ds(((jbase + k) * NROWS + row) * CHUNK, CHUNK)]

    def tail_slice():
        return x_hbm.at[pl.ds((30 * NROWS + row) * CHUNK, TAIL)]

    def start(k, buf, sem):
        pltpu.make_async_copy(chunk_slice(k), buf, sem).start()

    def wait(k, buf, sem):
        pltpu.make_async_copy(chunk_slice(k), buf, sem).wait()

    def run_pass(process_chunk, process_tail, carry_init):
        start(0, buf_a, sem_a)
        start(1, buf_b, sem_b)

        def outer(i, carry):
            for b, (buf, sem) in enumerate(((buf_a, sem_a), (buf_b, sem_b))):
                k = 2 * i + b
                wait(k, buf, sem)
                carry = process_chunk(buf, carry)

                @pl.when(k + 2 < K0)
                def _():
                    start(k + 2, buf, sem)
            return carry

        carry = lax.fori_loop(0, (K0 - 1) // 2, outer, carry_init)
        wait(K0 - 1, buf_a, sem_a)
        pltpu.make_async_copy(
            tail_slice(), buf_b.at[pl.ds(0, TAIL)], sem_b).start()
        carry = process_chunk(buf_a, carry)
        pltpu.make_async_copy(
            tail_slice(), buf_b.at[pl.ds(0, TAIL)], sem_b).wait()
        return process_tail(buf_b, carry)

    def p1_vregs(buf, carry, nv, unroll):
        def inner(j, cr):
            mn_v, mx_v = cr
            off = j * (16 * unroll)
            for u in range(unroll):
                v = buf[pl.ds(off + u * 16, 16)]
                mn_v = jnp.minimum(mn_v, v)
                mx_v = jnp.maximum(mx_v, v)
            return (mn_v, mx_v)

        return lax.fori_loop(0, nv // unroll, inner, carry)

    big = jnp.full((16,), jnp.inf, jnp.float32)
    mn_v, mx_v = run_pass(
        lambda buf, cr: p1_vregs(buf, cr, VPB, 16),
        lambda buf, cr: p1_vregs(buf, cr, VPT, 4),
        (big, -big))

    tmp_a[...] = mn_v
    tmp_b[...] = mx_v
    pltpu.sync_copy(tmp_a, sh_mm.at[s, 0])
    pltpu.sync_copy(tmp_b, sh_mm.at[s, 1])
    plsc.subcore_barrier()
    partner = jnp.bitwise_xor(s, 1)
    pltpu.sync_copy(sh_mm.at[partner, 0], tmp_a)
    pltpu.sync_copy(sh_mm.at[partner, 1], tmp_b)
    mn_v = jnp.minimum(mn_v, tmp_a[...])
    mx_v = jnp.maximum(mx_v, tmp_b[...])

    lane = lax.iota(jnp.int32, 16)
    for sh in (8, 4, 2, 1):
        perm = jnp.bitwise_xor(lane, sh)
        tmp_a[...] = mn_v
        tmp_b[...] = mx_v
        mn_v = jnp.minimum(mn_v, plsc.load_gather(tmp_a, [perm]))
        mx_v = jnp.maximum(mx_v, plsc.load_gather(tmp_b, [perm]))
    mn_b = mn_v
    mx_b = mx_v
    scale_b = float(NBINS) / (mx_b - mn_b)

    @pl.when(half == 0)
    def _():
        mmv = jnp.where(lane == 0, mn_b, jnp.where(lane == 1, mx_b, 0.0))
        tmp_a[...] = mmv
        pltpu.sync_copy(tmp_a, mm_hbm.at[row])

    zv = jnp.zeros((16,), jnp.float32)

    def zero(i, cr):
        hist_v[i, :] = zv
        return cr

    lax.fori_loop(0, NBINS, zero, 0)

    one_v = jnp.full((16,), 1.0, jnp.float32)
    lim_v = jnp.full((16,), NBINS - 1, jnp.int32)
    zero_i = jnp.zeros((16,), jnp.int32)

    def binloop(buf, nv, unroll):
        @plsc.parallel_loop(0, nv, unroll=unroll)
        def _(i):
            v = buf[pl.ds(i * 16, 16)]
            t = (v - mn_b) * scale_b
            idx = t.astype(jnp.int32)
            idx = jnp.maximum(jnp.minimum(idx, lim_v), zero_i)
            plsc.addupdate_scatter(hist_v, [idx, lane], one_v)

    def p2_tail(buf, carry):
        @pl.when(half == 1)
        def _():
            binloop(buf, VPT, 4)

        return carry

    run_pass(lambda buf, cr: (binloop(buf, VPB, 8), cr)[1], p2_tail, 0)

    pltpu.sync_copy(hist_v, hist_hbm.at[wid])


_sc_hist = functools.partial(
    pl.kernel,
    out_type=[
        jax.ShapeDtypeStruct((NWORKERS, NBINS, 16), jnp.float32),
        jax.ShapeDtypeStruct((NROWS, 16), jnp.float32),
    ],
    mesh=plsc.VectorSubcoreMesh(
        core_axis_name="c", subcore_axis_name="s",
        num_cores=NCORES, num_subcores=NSUB),
    compiler_params=pltpu.CompilerParams(
        needs_layout_passes=False, use_tc_tiling_on_sc=False),
    scratch_types=[
        pltpu.VMEM((CHUNK,), jnp.float32),
        pltpu.VMEM((CHUNK,), jnp.float32),
        pltpu.VMEM((NBINS, 16), jnp.float32),
        pltpu.VMEM((16,), jnp.float32),
        pltpu.VMEM((16,), jnp.float32),
        pltpu.VMEM_SHARED((NSUB, 2, 16), jnp.float32),
        pltpu.SemaphoreType.DMA,
        pltpu.SemaphoreType.DMA,
    ],
)(_sc_body)


def _tc_body(hist_ref, mm_ref, wc_ref, wb_ref, b_ref, out_ref):
    h = hist_ref[...].reshape(NROWS, 2, NBINS, 16)
    counts = jnp.sum(h, axis=(1, 3))
    total = jnp.sum(counts, axis=1, keepdims=True)
    counts = counts / total
    mn = mm_ref[:, 0:1]
    mx = mm_ref[:, 1:2]
    k = lax.broadcasted_iota(jnp.int32, (NROWS, NBINS + 1), 1).astype(jnp.float32)
    bounds = mn + (mx - mn) * (k * (1.0 / NBINS))
    acc = lax.dot_general(counts, wc_ref[...], (((1,), (0,)), ((), ())),
                          preferred_element_type=jnp.float32)
    acc += lax.dot_general(bounds, wb_ref[...], (((1,), (0,)), ((), ())),
                           preferred_element_type=jnp.float32)
    out_ref[...] = acc + b_ref[...][None, :]


def _tc_linear(hist, mm, wc, wb, b):
    return pl.pallas_call(
        _tc_body,
        out_shape=jax.ShapeDtypeStruct((NROWS, NOUT), jnp.float32),
    )(hist, mm, wc, wb, b)


def kernel(x, W, b):
    xf = _linz(x)
    hist, mm = _sc_hist(xf)
    wc = W[:, :NBINS].T
    wb = W[:, NBINS:].T
    return _tc_linear(hist, mm, wc, wb, b)
